# Initial kernel scaffold; baseline (speedup 1.0000x reference)
import jax
import jax.numpy as jnp
from jax.experimental import pallas as pl


def kernel(batch_data, table):
    raise NotImplementedError("write your pallas kernel here")


# speedup vs baseline: 1.4750x; 1.4750x over previous
import functools

import jax
import jax.numpy as jnp
from jax import lax
from jax.experimental import pallas as pl
from jax.experimental.pallas import tpu as pltpu
from jax.experimental.pallas import tpu_sc as plsc


def _gather_sc(idx, table, n_per_w, chunk):
    N = idx.shape[0]
    D = table.shape[1]
    info = plsc.get_sparse_core_info()
    num_cores = info.num_cores
    n_chunks = n_per_w // chunk

    mesh = plsc.VectorSubcoreMesh(core_axis_name="c", subcore_axis_name="s")

    @functools.partial(
        pl.kernel,
        out_type=jax.ShapeDtypeStruct((N, D), jnp.float32),
        mesh=mesh,
        scratch_types=[
            pltpu.VMEM((chunk,), jnp.int32),
            pltpu.VMEM((chunk, D), jnp.float32),
            pltpu.SemaphoreType.DMA,
        ],
        compiler_params=pltpu.CompilerParams(use_tc_tiling_on_sc=False),
    )
    def body(idx_hbm, table_hbm, out_hbm, idx_v, rows_v, sem):
        wid = lax.axis_index("s") * num_cores + lax.axis_index("c")
        base = wid * n_per_w

        def step(g, carry):
            start = base + g * chunk
            pltpu.sync_copy(idx_hbm.at[pl.ds(start, chunk)], idx_v)
            pltpu.async_copy(table_hbm.at[idx_v], rows_v, sem).wait()
            pltpu.sync_copy(rows_v, out_hbm.at[pl.ds(start, chunk)])
            return carry

        lax.fori_loop(0, n_chunks, step, 0)

    return body(idx, table)


def kernel(batch_data, table):
    B, H = batch_data.shape
    D = table.shape[1]
    N = B * H
    info = plsc.get_sparse_core_info()
    nw = info.num_cores * info.num_subcores
    n_per_w = N // nw
    out = _gather_sc(batch_data.reshape(N), table, n_per_w, chunk=1600)
    return out.reshape(B, H, D)


# speedup vs baseline: 1.4998x; 1.0168x over previous
import functools

import jax
import jax.numpy as jnp
from jax import lax
from jax.experimental import pallas as pl
from jax.experimental.pallas import tpu as pltpu
from jax.experimental.pallas import tpu_sc as plsc

_NBUF = 2


def _gather_sc(idx, table, n_per_w, chunk):
    N = idx.shape[0]
    D = table.shape[1]
    info = plsc.get_sparse_core_info()
    num_cores = info.num_cores
    n_chunks = n_per_w // chunk
    assert n_chunks % _NBUF == 0 and n_chunks >= 2 * _NBUF

    mesh = plsc.VectorSubcoreMesh(core_axis_name="c", subcore_axis_name="s")

    @functools.partial(
        pl.kernel,
        out_type=jax.ShapeDtypeStruct((N, D), jnp.float32),
        mesh=mesh,
        scratch_types=[
            pltpu.VMEM((n_per_w,), jnp.int32),
            [pltpu.VMEM((chunk, D), jnp.float32) for _ in range(_NBUF)],
            [pltpu.SemaphoreType.DMA for _ in range(_NBUF)],
            [pltpu.SemaphoreType.DMA for _ in range(_NBUF)],
        ],
        compiler_params=pltpu.CompilerParams(use_tc_tiling_on_sc=False),
    )
    def body(idx_hbm, table_hbm, out_hbm, idx_all, rows, gsem, wsem):
        wid = lax.axis_index("s") * num_cores + lax.axis_index("c")
        base = wid * n_per_w
        pltpu.sync_copy(idx_hbm.at[pl.ds(base, n_per_w)], idx_all)

        def gather_start(g, b):
            pltpu.make_async_copy(
                table_hbm.at[idx_all.at[pl.ds(g * chunk, chunk)]],
                rows[b], gsem[b],
            ).start()

        def retire(g, b):
            pltpu.make_async_copy(
                table_hbm.at[idx_all.at[pl.ds(0, chunk)]], rows[b], gsem[b]
            ).wait()
            pltpu.make_async_copy(
                rows[b], out_hbm.at[pl.ds(base + g * chunk, chunk)], wsem[b]
            ).start()

        for g in range(_NBUF):
            gather_start(g, g)
            if g >= 1:
                retire(g - 1, g - 1)

        @pl.loop(_NBUF, n_chunks, step=_NBUF)
        def _(g0):
            for b in range(_NBUF):
                g = g0 + b
                pb = (b - 1) % _NBUF
                pltpu.make_async_copy(
                    rows[b], out_hbm.at[pl.ds(base, chunk)], wsem[b]
                ).wait()
                gather_start(g, b)
                retire(g - 1, pb)

        retire(n_chunks - 1, (n_chunks - 1) % _NBUF)
        for b in range(_NBUF):
            pltpu.make_async_copy(
                rows[b], out_hbm.at[pl.ds(base, chunk)], wsem[b]
            ).wait()

    return body(idx, table)


def kernel(batch_data, table):
    B, H = batch_data.shape
    D = table.shape[1]
    N = B * H
    info = plsc.get_sparse_core_info()
    nw = info.num_cores * info.num_subcores
    n_per_w = N // nw
    out = _gather_sc(batch_data.reshape(N), table, n_per_w, chunk=1600)
    return out.reshape(B, H, D)
